# Initial kernel scaffold; baseline (speedup 1.0000x reference)
import jax
import jax.numpy as jnp
from jax.experimental import pallas as pl


def kernel(tokens, row_indices, col_indices, token_table, row_table, col_table):
    raise NotImplementedError("write your pallas kernel here")


# speedup vs baseline: 2.6813x; 2.6813x over previous
import functools

import jax
import jax.numpy as jnp
from jax import lax
from jax.experimental import pallas as pl
from jax.experimental.pallas import tpu as pltpu
from jax.experimental.pallas import tpu_sc as plsc

B = 4096
L = 200
D = 64
T = B * L

NW = 32
PER_W = T // NW
C = 128
N_CHUNKS = PER_W // C

_mesh = plsc.VectorSubcoreMesh(core_axis_name="c", subcore_axis_name="s")


@functools.partial(
    pl.kernel,
    mesh=_mesh,
    compiler_params=pltpu.CompilerParams(use_tc_tiling_on_sc=False),
    out_type=jax.ShapeDtypeStruct((T, D), jnp.float32),
    scratch_types=[
        pltpu.VMEM((C,), jnp.int32),
        pltpu.VMEM((C,), jnp.int32),
        pltpu.VMEM((C,), jnp.int32),
        pltpu.VMEM((C, D), jnp.float32),
        pltpu.VMEM((C, D), jnp.float32),
        pltpu.VMEM((C, D), jnp.float32),
        pltpu.SemaphoreType.DMA,
        pltpu.SemaphoreType.DMA,
        pltpu.SemaphoreType.DMA,
    ],
)
def _emb_lookup(tok_hbm, row_hbm, col_hbm, ttab, rtab, ctab, out_hbm,
                idx_t, idx_r, idx_c, buf_t, buf_r, buf_c, s0, s1, s2):
    wid = lax.axis_index("s") * 2 + lax.axis_index("c")
    base0 = wid * PER_W

    def chunk_body(ci, carry):
        base = base0 + ci * C
        pltpu.sync_copy(tok_hbm.at[pl.ds(base, C)], idx_t)
        pltpu.sync_copy(row_hbm.at[pl.ds(base, C)], idx_r)
        pltpu.sync_copy(col_hbm.at[pl.ds(base, C)], idx_c)
        ct = pltpu.async_copy(ttab.at[idx_t], buf_t, s0)
        cr = pltpu.async_copy(rtab.at[idx_r], buf_r, s1)
        cc = pltpu.async_copy(ctab.at[idx_c], buf_c, s2)
        ct.wait()
        cr.wait()
        cc.wait()

        def row_body(i, c2):
            for d in range(D // 16):
                sl = pl.ds(d * 16, 16)
                buf_t[i, sl] = buf_t[i, sl] + buf_r[i, sl] + buf_c[i, sl]
            return c2

        lax.fori_loop(0, C, row_body, 0, unroll=2)
        pltpu.sync_copy(buf_t, out_hbm.at[pl.ds(base, C)])
        return carry

    lax.fori_loop(0, N_CHUNKS, chunk_body, 0)


def kernel(tokens, row_indices, col_indices, token_table, row_table, col_table):
    tok = tokens.reshape(T).astype(jnp.int32)
    ri = row_indices.reshape(T).astype(jnp.int32)
    ci = col_indices.reshape(T).astype(jnp.int32)
    out = _emb_lookup(tok, ri, ci, token_table, row_table, col_table)
    return out.reshape(B, L, D)


# speedup vs baseline: 3.7763x; 1.4084x over previous
import functools

import jax
import jax.numpy as jnp
from jax import lax
from jax.experimental import pallas as pl
from jax.experimental.pallas import tpu as pltpu
from jax.experimental.pallas import tpu_sc as plsc

B = 4096
L = 200
D = 64
T = B * L

NW = 32
PER_W = T // NW
C = 128
NSLOT = 4
MACRO = PER_W // (C * NSLOT)

_mesh = plsc.VectorSubcoreMesh(core_axis_name="c", subcore_axis_name="s")


@functools.partial(
    pl.kernel,
    mesh=_mesh,
    compiler_params=pltpu.CompilerParams(use_tc_tiling_on_sc=False),
    out_type=jax.ShapeDtypeStruct((T, D), jnp.float32),
    scratch_types=[
        pltpu.VMEM((NSLOT, C), jnp.int32),
        pltpu.VMEM((NSLOT, C), jnp.int32),
        pltpu.VMEM((NSLOT, C), jnp.int32),
        pltpu.VMEM((NSLOT, C, D), jnp.float32),
        pltpu.VMEM((NSLOT, C, D), jnp.float32),
        pltpu.VMEM((NSLOT, C, D), jnp.float32),
    ]
    + [pltpu.SemaphoreType.DMA] * (3 * NSLOT),
)
def _emb_lookup(tok_hbm, row_hbm, col_hbm, ttab, rtab, ctab, out_hbm,
                idx_t, idx_r, idx_c, buf_t, buf_r, buf_c, *sems):
    s_idx = sems[0:NSLOT]
    s_gat = sems[NSLOT:2 * NSLOT]
    s_out = sems[2 * NSLOT:3 * NSLOT]
    wid = lax.axis_index("s") * 2 + lax.axis_index("c")
    base0 = wid * PER_W

    def issue_idx(j, chunk):
        src = pl.ds(base0 + chunk * C, C)
        pltpu.async_copy(tok_hbm.at[src], idx_t.at[j], s_idx[j])
        pltpu.async_copy(row_hbm.at[src], idx_r.at[j], s_idx[j])
        pltpu.async_copy(col_hbm.at[src], idx_c.at[j], s_idx[j])

    def wait_idx(j):
        pltpu.make_async_copy(tok_hbm.at[pl.ds(0, C)], idx_t.at[j], s_idx[j]).wait()
        pltpu.make_async_copy(row_hbm.at[pl.ds(0, C)], idx_r.at[j], s_idx[j]).wait()
        pltpu.make_async_copy(col_hbm.at[pl.ds(0, C)], idx_c.at[j], s_idx[j]).wait()

    def issue_gathers(j):
        pltpu.async_copy(ttab.at[idx_t.at[j]], buf_t.at[j], s_gat[j])
        pltpu.async_copy(rtab.at[idx_r.at[j]], buf_r.at[j], s_gat[j])
        pltpu.async_copy(ctab.at[idx_c.at[j]], buf_c.at[j], s_gat[j])

    def wait_gathers(j):
        pltpu.make_async_copy(ttab.at[idx_t.at[j]], buf_t.at[j], s_gat[j]).wait()
        pltpu.make_async_copy(rtab.at[idx_r.at[j]], buf_r.at[j], s_gat[j]).wait()
        pltpu.make_async_copy(ctab.at[idx_c.at[j]], buf_c.at[j], s_gat[j]).wait()

    def issue_out(j, chunk):
        dst = pl.ds(base0 + chunk * C, C)
        pltpu.async_copy(buf_t.at[j], out_hbm.at[dst], s_out[j])

    def wait_out(j):
        pltpu.make_async_copy(buf_t.at[j], out_hbm.at[pl.ds(0, C)], s_out[j]).wait()

    for j in range(NSLOT):
        issue_idx(j, j)
    for j in range(NSLOT):
        wait_idx(j)
        issue_gathers(j)

    def macro_body(m, carry):
        chunk0 = m * NSLOT
        for j in range(NSLOT):
            wait_gathers(j)

            def row_body(i, c2):
                for dd in range(D // 16):
                    sl = pl.ds(dd * 16, 16)
                    v = buf_r[j, i, sl] + buf_c[j, i, sl]
                    plsc.addupdate(buf_t.at[j, i, sl], v)
                return c2

            lax.fori_loop(0, C, row_body, 0, unroll=2)
            issue_out(j, chunk0 + j)
            @pl.when(m < MACRO - 1)
            def _():
                issue_idx(j, chunk0 + NSLOT + j)

        @pl.when(m < MACRO - 1)
        def _():
            for j in range(NSLOT):
                wait_idx(j)
                wait_out(j)
                issue_gathers(j)

        return carry

    lax.fori_loop(0, MACRO, macro_body, 0)
    for j in range(NSLOT):
        wait_out(j)


def kernel(tokens, row_indices, col_indices, token_table, row_table, col_table):
    tok = tokens.reshape(T).astype(jnp.int32)
    ri = row_indices.reshape(T).astype(jnp.int32)
    ci = col_indices.reshape(T).astype(jnp.int32)
    out = _emb_lookup(tok, ri, ci, token_table, row_table, col_table)
    return out.reshape(B, L, D)
